# Initial kernel scaffold; baseline (speedup 1.0000x reference)
import jax
import jax.numpy as jnp
from jax.experimental import pallas as pl


def kernel(word_logits, word_labels, segment_masks, phoneme_nums, segment_nums):
    raise NotImplementedError("write your pallas kernel here")


# speedup vs baseline: 7.3094x; 7.3094x over previous
import jax
import jax.numpy as jnp
from jax.experimental import pallas as pl
from jax.experimental.pallas import tpu as pltpu


def _loss_body(x_ref, lab_ref, mask_ref, out_ref):
    x = x_ref[:, 0, :]
    B, V = x.shape
    m = jnp.max(x, axis=1, keepdims=True)
    s = jnp.sum(jnp.exp(x - m), axis=1, keepdims=True)
    lse = m + jnp.log(s)
    lab = lab_ref[:]
    col = jax.lax.broadcasted_iota(jnp.int32, (B, V), 1)
    xg = jnp.sum(jnp.where(col == lab, x, 0.0), axis=1, keepdims=True)
    loss = (lse - xg) * mask_ref[:, 0:1]
    out_ref[:, :] = jnp.mean(loss, keepdims=True)


def kernel(word_logits, word_labels, segment_masks, phoneme_nums, segment_nums):
    B, S, V = word_logits.shape
    lab2d = word_labels.reshape(B, 1)
    out = pl.pallas_call(
        _loss_body,
        grid=(1,),
        in_specs=[
            pl.BlockSpec((B, 8, V), lambda i: (0, 0, 0)),
            pl.BlockSpec((B, 1), lambda i: (0, 0)),
            pl.BlockSpec((B, S), lambda i: (0, 0)),
        ],
        out_specs=pl.BlockSpec((1, 1), lambda i: (0, 0)),
        out_shape=jax.ShapeDtypeStruct((1, 1), jnp.float32),
    )(word_logits, lab2d, segment_masks)
    return out[0, 0]
